# baseline (speedup 1.0000x reference)
import functools

import jax
import jax.numpy as jnp
from jax import lax
from jax.experimental import pallas as pl
from jax.experimental.pallas import tpu as pltpu
from jax.experimental.pallas import tpu_sc as plsc


def _tr_body(lo_ref, hi_ref, out_ref):
    d = lo_ref.shape[0]
    out_ref[:, 0:d] = lo_ref[...].T
    out_ref[:, d:2 * d] = hi_ref[...].T


def _make_transpose(Dd, V, BT):
    S = BT * (V // (2 * BT))
    Vh = V - S
    nb_lo = S // BT
    return pl.pallas_call(
        _tr_body,
        grid=(pl.cdiv(Vh, BT),),
        in_specs=[
            pl.BlockSpec((Dd, BT), lambda i: (0, i)),
            pl.BlockSpec((Dd, BT), lambda i: (0, i + nb_lo)),
        ],
        out_specs=pl.BlockSpec((BT, 2 * Dd), lambda i: (i, 0)),
        out_shape=jax.ShapeDtypeStruct((Vh, 2 * Dd), jnp.float32),
        compiler_params=pltpu.CompilerParams(vmem_limit_bytes=112 * 1024 * 1024),
    ), S, Vh


def _make_gather2(N, D, C, NC, NS):
    NW = NC * NS
    per_w = N // NW
    n_chunks = per_w // C
    mesh = plsc.VectorSubcoreMesh(core_axis_name="c", subcore_axis_name="s")

    @functools.partial(
        pl.kernel,
        mesh=mesh,
        out_type=jax.ShapeDtypeStruct((N, 2 * D), jnp.float32),
        scratch_types=[
            pltpu.VMEM((per_w,), jnp.int32),
            pltpu.VMEM((per_w,), jnp.int32),
            pltpu.VMEM((2, C, D), jnp.float32),
            pltpu.VMEM((2, C, D), jnp.float32),
            pltpu.SemaphoreType.DMA,
            pltpu.SemaphoreType.DMA,
            pltpu.SemaphoreType.DMA,
            pltpu.SemaphoreType.DMA,
        ],
        compiler_params=pltpu.CompilerParams(use_tc_tiling_on_sc=False),
    )
    def gather2(word_hbm, entity_hbm, wid_hbm, eid_hbm, out_hbm,
                widx_v, eidx_v, wbuf, ebuf, gsem0, gsem1, ssem0, ssem1):
        wid = lax.axis_index("s") * NC + lax.axis_index("c")
        base = wid * per_w
        gsem = (gsem0, gsem1)
        ssem = (ssem0, ssem1)

        pltpu.sync_copy(wid_hbm.at[pl.ds(base, per_w)], widx_v)
        pltpu.sync_copy(eid_hbm.at[pl.ds(base, per_w)], eidx_v)

        def issue_gather(i, p):
            return (
                pltpu.async_copy(
                    word_hbm.at[widx_v.at[pl.ds(i * C, C)]], wbuf.at[p], gsem[p]),
                pltpu.async_copy(
                    entity_hbm.at[eidx_v.at[pl.ds(i * C, C)]], ebuf.at[p], gsem[p]),
            )

        def issue_scatter(i, p):
            start = base + i * C
            return (
                pltpu.async_copy(
                    wbuf.at[p], out_hbm.at[pl.ds(start, C), pl.ds(0, D)], ssem[p]),
                pltpu.async_copy(
                    ebuf.at[p], out_hbm.at[pl.ds(start, C), pl.ds(D, D)], ssem[p]),
            )

        g = [None, None]
        sc = [None, None]
        g[0] = issue_gather(0, 0)
        for i in range(n_chunks):
            p = i % 2
            g[p][0].wait()
            g[p][1].wait()
            sc[p] = issue_scatter(i, p)
            if i + 1 < n_chunks:
                q = 1 - p
                if sc[q] is not None:
                    sc[q][0].wait()
                    sc[q][1].wait()
                g[q] = issue_gather(i + 1, q)
        for p in (0, 1):
            if sc[p] is not None:
                sc[p][0].wait()
                sc[p][1].wait()

    return gather2


def kernel(word_table, entity_table, word_ids, entity_ids):
    B, H = word_ids.shape
    D = word_table.shape[1]
    N = B * H
    info = plsc.get_sparse_core_info()
    NC, NS = info.num_cores, info.num_subcores
    C = 400
    VW = word_table.shape[0]
    VE = entity_table.shape[0]
    wT = word_table.T
    eT = entity_table.T
    tr_w, SW, VhW = _make_transpose(D, VW, 24576)
    tr_e, SE, VhE = _make_transpose(D, VE, 8192)
    word_rm = tr_w(wT, wT).reshape(2 * VhW, D)
    entity_rm = tr_e(eT, eT).reshape(2 * VhE, D)
    wv = word_ids.T.reshape(N).astype(jnp.int32)
    ev = entity_ids.T.reshape(N).astype(jnp.int32)
    wid_flat = jnp.where(wv < SW, 2 * wv, 2 * (wv - SW) + 1)
    eid_flat = jnp.where(ev < SE, 2 * ev, 2 * (ev - SE) + 1)
    out = _make_gather2(N, D, C, NC, NS)(word_rm, entity_rm, wid_flat, eid_flat)
    return out.reshape(H, B, 2 * D).transpose(1, 0, 2)
